# baseline (speedup 1.0000x reference)
import functools

import jax
import jax.numpy as jnp
from jax import lax
from jax.experimental import pallas as pl
from jax.experimental.pallas import tpu as pltpu
from jax.experimental.pallas import tpu_sc as plsc

NC = 2
NS = 16
L = 16
NW = NC * NS

C = 32
PIECE = 80


def _sc_scores_kernel(B, P, D):
    assert B % NW == 0
    bpw = B // NW
    assert bpw % C == 0
    n_chunks = bpw // C
    cp = C * P
    n_pieces = -(-cp // PIECE)
    assert cp % PIECE == 0
    n_seg = D // L

    mesh = plsc.VectorSubcoreMesh(
        core_axis_name="c", subcore_axis_name="s",
        num_cores=NC, num_subcores=NS)

    @functools.partial(
        pl.kernel,
        out_type=jax.ShapeDtypeStruct((B * P,), jnp.float32),
        mesh=mesh,
        compiler_params=pltpu.CompilerParams(needs_layout_passes=False),
        scratch_types=[
            pltpu.VMEM((C,), jnp.int32),
            pltpu.VMEM((cp,), jnp.int32),
            pltpu.VMEM((C,), jnp.int32),
            pltpu.VMEM((cp,), jnp.int32),
            pltpu.VMEM((C, 2 * D), jnp.float32),
            pltpu.VMEM((cp, 2 * D), jnp.float32),
            pltpu.VMEM((cp,), jnp.float32),
            pltpu.SemaphoreType.DMA,
        ],
    )
    def sc_kernel(pinpar_hbm, cidxpar_hbm, pinp_hbm, cidxp_hbm, w_hbm, v_hbm,
                  out_hbm, pinp_v, cidxp_v, pinpar_v, cidxpar_v,
                  in_rows, ctx_rows, scores_v, sem):
        wid = lax.axis_index("s") * NC + lax.axis_index("c")
        base_b = wid * bpw
        zeros = jnp.zeros((L,), jnp.float32)

        def chunk_body(ci, _):
            b0 = base_b + ci * C
            p0 = b0 * P
            pltpu.sync_copy(pinp_hbm.at[pl.ds(b0, C)], pinp_v)
            pltpu.sync_copy(cidxp_hbm.at[pl.ds(p0, cp)], cidxp_v)
            pltpu.sync_copy(pinpar_hbm.at[pl.ds(b0, C)], pinpar_v)
            pltpu.sync_copy(cidxpar_hbm.at[pl.ds(p0, cp)], cidxpar_v)
            copies = [pltpu.async_copy(w_hbm.at[pinp_v], in_rows, sem)]
            for k in range(n_pieces):
                copies.append(pltpu.async_copy(
                    v_hbm.at[cidxp_v.at[pl.ds(k * PIECE, PIECE)]],
                    ctx_rows.at[pl.ds(k * PIECE, PIECE), :], sem))
            for cpy in copies:
                cpy.wait()

            def zero_body(k, _):
                scores_v[pl.ds(k * L, L)] = zeros
                return 0

            lax.fori_loop(0, cp // L, zero_body, 0)

            iota16 = lax.iota(jnp.int32, L)

            def row_body(i, _):
                ifull = jnp.full((L,), i, jnp.int32)
                ipar = plsc.load_gather(pinpar_v, [ifull])
                icb = ipar * D + iota16
                segs = [plsc.load_gather(in_rows, [ifull, icb + s * L])
                        for s in range(n_seg)]
                r0 = i * P
                for j in range(P):
                    r = r0 + j
                    rfull = jnp.full((L,), r, jnp.int32)
                    cpar = plsc.load_gather(cidxpar_v, [rfull])
                    ccb = cpar * D + iota16
                    acc = plsc.load_gather(ctx_rows, [rfull, ccb]) * segs[0]
                    for s in range(1, n_seg):
                        acc = acc + plsc.load_gather(
                            ctx_rows, [rfull, ccb + s * L]) * segs[s]
                    plsc.addupdate_scatter(scores_v, [rfull], acc)
                return 0

            lax.fori_loop(0, C, row_body, 0)
            pltpu.sync_copy(scores_v, out_hbm.at[pl.ds(p0, cp)])
            return 0

        lax.fori_loop(0, n_chunks, chunk_body, 0)

    return sc_kernel


def _tc_loss_kernel(scores_ref, out_ref, *, n_pos, P, B):
    x = scores_ref[...]
    rows, cols = x.shape
    flat = (lax.broadcasted_iota(jnp.int32, (rows, cols), 0) * cols
            + lax.broadcasted_iota(jnp.int32, (rows, cols), 1))
    is_pos = (flat % P) < n_pos
    y = jnp.clip(x, -10.0, 10.0)
    t = jnp.where(is_pos, -y, y)
    f = jnp.log1p(jnp.exp(t))
    out_ref[0, 0] = jnp.sum(f) * (1.0 / B)


def kernel(W, V, pos_input, pos_target, neg):
    B = pos_input.shape[0]
    n_pos = pos_target.shape[1]
    n_neg = neg.shape[1]
    P = n_pos + n_neg
    D = W.shape[1]
    n_words = W.shape[0]

    Wg = W.reshape(n_words // 2, 2 * D)
    Vg = V.reshape(n_words // 2, 2 * D)
    ctx_idx = jnp.concatenate([pos_target, neg], axis=1).reshape(-1)
    ctx_phys = ctx_idx >> 1
    pin_phys = pos_input >> 1
    ctx_par = ctx_idx & 1
    pin_par = pos_input & 1

    sc = _sc_scores_kernel(B, P, D)
    scores = sc(pin_par, ctx_par, pin_phys, ctx_phys, Wg, Vg)

    total = B * P
    cols = 128
    rows = total // cols
    scores2d = scores.reshape(rows, cols)

    out = pl.pallas_call(
        functools.partial(_tc_loss_kernel, n_pos=n_pos, P=P, B=B),
        out_shape=jax.ShapeDtypeStruct((1, 1), jnp.float32),
        in_specs=[pl.BlockSpec(memory_space=pltpu.VMEM)],
        out_specs=pl.BlockSpec(memory_space=pltpu.SMEM),
    )(scores2d)
    return out[0, 0]
